# baseline (speedup 1.0000x reference)
import functools

import jax
import jax.numpy as jnp
from jax import lax
from jax.experimental import pallas as pl
from jax.experimental.pallas import tpu as pltpu
from jax.experimental.pallas import tpu_sc as plsc

N_NODES = 100000
N_EDGES = 3200000
IN_DIM = 128
HID = 16
OUT_DIM = 128

NC = 2
NS = 16
NW = NC * NS
EPW = N_EDGES // NW
CHUNK = 80
NCHUNKS = EPW // CHUNK
N_PAD = 100352
ROWS_PER_TILE = N_PAD // NS

_f32 = jnp.float32


def _mlp_in_body(x_ref, w_ref, b_ref, g_ref, n_ref):
    h = jnp.dot(x_ref[...], w_ref[...], preferred_element_type=_f32)
    h = jnp.maximum(h + b_ref[...], 0.0)
    n = jnp.sqrt(jnp.sum(h * h, axis=1, keepdims=True))
    g_ref[...] = h / jnp.maximum(n, 1e-12)
    n_ref[...] = jnp.broadcast_to(n, (n.shape[0], 8))


def _mlp_in(x, W1, b1):
    B = 2000
    return pl.pallas_call(
        _mlp_in_body,
        grid=(N_NODES // B,),
        in_specs=[
            pl.BlockSpec((B, IN_DIM), lambda i: (i, 0)),
            pl.BlockSpec((IN_DIM, HID), lambda i: (0, 0)),
            pl.BlockSpec((1, HID), lambda i: (0, 0)),
        ],
        out_specs=[
            pl.BlockSpec((B, HID), lambda i: (i, 0)),
            pl.BlockSpec((B, 8), lambda i: (i, 0)),
        ],
        out_shape=[
            jax.ShapeDtypeStruct((N_PAD, HID), _f32),
            jax.ShapeDtypeStruct((N_PAD, 8), _f32),
        ],
    )(x, W1, b1.reshape(1, HID))


def _conv_body(g_hbm, n_hbm, src_hbm, dst_hbm, beta_hbm, z16_hbm, z1_hbm,
               accp_hbm, denp_hbm,
               acc_sh, den_sh, betav, sidv, didv, srows, drows, nsv, wv, sem):
    cid = lax.axis_index("c")
    sid = lax.axis_index("s")
    wid = sid * NC + cid

    row0 = sid * ROWS_PER_TILE
    pltpu.sync_copy(z16_hbm.at[pl.ds(row0, ROWS_PER_TILE)],
                    acc_sh.at[pl.ds(row0, ROWS_PER_TILE)])
    pltpu.sync_copy(z1_hbm.at[pl.ds(row0, ROWS_PER_TILE)],
                    den_sh.at[pl.ds(row0, ROWS_PER_TILE)])
    pltpu.sync_copy(beta_hbm, betav)
    plsc.subcore_barrier()

    beta = betav[...]
    ebase = wid * EPW
    zcol = jnp.zeros((16,), jnp.int32)

    def chunk_body(i, carry):
        base = ebase + i * CHUNK
        pltpu.sync_copy(src_hbm.at[pl.ds(base, CHUNK)], sidv)
        pltpu.sync_copy(dst_hbm.at[pl.ds(base, CHUNK)], didv)
        pltpu.async_copy(g_hbm.at[sidv], srows, sem).wait()
        pltpu.async_copy(g_hbm.at[didv], drows, sem).wait()
        pltpu.async_copy(n_hbm.at[sidv], nsv, sem).wait()
        for g in range(CHUNK // 16):
            eidx = lax.iota(jnp.int32, 16) + g * 16
            dot = jnp.zeros((16,), _f32)
            scols = []
            for f in range(HID):
                fv = jnp.full((16,), f, jnp.int32)
                sf = plsc.load_gather(srows, [eidx, fv])
                df = plsc.load_gather(drows, [eidx, fv])
                scols.append(sf)
                dot = dot + sf * df
            w = jnp.exp(dot * beta)
            wv[pl.ds(g * 16, 16)] = w
            wn = w * plsc.load_gather(nsv, [eidx, zcol])
            for f in range(HID):
                fv = jnp.full((16,), f, jnp.int32)
                plsc.store_scatter(srows, [eidx, fv], scols[f] * wn)
        pltpu.sync_copy(srows, acc_sh.at[didv], add=True)
        pltpu.sync_copy(wv, den_sh.at[didv], add=True)
        return carry

    lax.fori_loop(0, NCHUNKS, chunk_body, 0)
    plsc.subcore_barrier()
    pltpu.sync_copy(acc_sh.at[pl.ds(row0, ROWS_PER_TILE)],
                    accp_hbm.at[cid, pl.ds(row0, ROWS_PER_TILE)])
    pltpu.sync_copy(den_sh.at[pl.ds(row0, ROWS_PER_TILE)],
                    denp_hbm.at[cid, 0, pl.ds(row0, ROWS_PER_TILE)])


def _conv_edges(g, n, src, dst, beta_vec, z16, z1):
    mesh = plsc.VectorSubcoreMesh(core_axis_name="c", subcore_axis_name="s")
    fn = pl.kernel(
        _conv_body,
        mesh=mesh,
        compiler_params=pltpu.CompilerParams(
            needs_layout_passes=False, use_tc_tiling_on_sc=False),
        out_type=[
            jax.ShapeDtypeStruct((NC, N_PAD, HID), _f32),
            jax.ShapeDtypeStruct((NC, 1, N_PAD), _f32),
        ],
        scratch_types=[
            pltpu.VMEM_SHARED((N_PAD, HID), _f32),
            pltpu.VMEM_SHARED((N_PAD,), _f32),
            pltpu.VMEM((16,), _f32),
            pltpu.VMEM((CHUNK,), jnp.int32),
            pltpu.VMEM((CHUNK,), jnp.int32),
            pltpu.VMEM((CHUNK, HID), _f32),
            pltpu.VMEM((CHUNK, HID), _f32),
            pltpu.VMEM((CHUNK, 8), _f32),
            pltpu.VMEM((CHUNK,), _f32),
            pltpu.SemaphoreType.DMA,
        ],
    )
    return fn(g, n, src, dst, beta_vec, z16, z1)


def _combine_core(accp_ref, denp_ref, g_ref, n_ref, beta_ref):
    g = g_ref[...]
    n = n_ref[...][:, :1]
    rr = 1.0 / jnp.maximum(n, 1e-12)
    beta = beta_ref[0, 0]
    wl = jnp.exp((n * rr) * (n * rr) * beta)
    num = accp_ref[0] + accp_ref[1] + wl * n * g
    den = denp_ref[0, 0] + denp_ref[1, 0] + wl[:, 0]
    return num / jnp.maximum(den, 1e-16)[:, None]


def _combine_body(accp_ref, denp_ref, g_ref, n_ref, beta_ref,
                  g2_ref, n2_ref):
    h2 = _combine_core(accp_ref, denp_ref, g_ref, n_ref, beta_ref)
    n2 = jnp.sqrt(jnp.sum(h2 * h2, axis=1, keepdims=True))
    g2_ref[...] = h2 / jnp.maximum(n2, 1e-12)
    n2_ref[...] = jnp.broadcast_to(n2, (n2.shape[0], 8))


def _final_body(accp_ref, denp_ref, g_ref, n_ref, beta_ref, w2_ref, b2_ref,
                o_ref):
    h2 = _combine_core(accp_ref, denp_ref, g_ref, n_ref, beta_ref)
    o = jnp.dot(h2, w2_ref[...], preferred_element_type=_f32) + b2_ref[...]
    o_ref[...] = jnp.tanh(o)


def _combine(accp, denp, g, n, beta11):
    B = 2048
    return pl.pallas_call(
        _combine_body,
        grid=(N_PAD // B,),
        in_specs=[
            pl.BlockSpec((NC, B, HID), lambda i: (0, i, 0)),
            pl.BlockSpec((NC, 1, B), lambda i: (0, 0, i)),
            pl.BlockSpec((B, HID), lambda i: (i, 0)),
            pl.BlockSpec((B, 8), lambda i: (i, 0)),
            pl.BlockSpec((1, 1), lambda i: (0, 0)),
        ],
        out_specs=[
            pl.BlockSpec((B, HID), lambda i: (i, 0)),
            pl.BlockSpec((B, 8), lambda i: (i, 0)),
        ],
        out_shape=[
            jax.ShapeDtypeStruct((N_PAD, HID), _f32),
            jax.ShapeDtypeStruct((N_PAD, 8), _f32),
        ],
    )(accp, denp, g, n, beta11)


def _final(accp, denp, g, n, beta11, W2, b2):
    B = 2048
    return pl.pallas_call(
        _final_body,
        grid=(N_PAD // B,),
        in_specs=[
            pl.BlockSpec((NC, B, HID), lambda i: (0, i, 0)),
            pl.BlockSpec((NC, 1, B), lambda i: (0, 0, i)),
            pl.BlockSpec((B, HID), lambda i: (i, 0)),
            pl.BlockSpec((B, 8), lambda i: (i, 0)),
            pl.BlockSpec((1, 1), lambda i: (0, 0)),
            pl.BlockSpec((HID, OUT_DIM), lambda i: (0, 0)),
            pl.BlockSpec((1, OUT_DIM), lambda i: (0, 0)),
        ],
        out_specs=pl.BlockSpec((B, OUT_DIM), lambda i: (i, 0)),
        out_shape=jax.ShapeDtypeStruct((N_PAD, OUT_DIM), _f32),
    )(accp, denp, g, n, beta11, W2, b2.reshape(1, OUT_DIM))


def kernel(x, edge_index, W1, b1, W2, b2, beta2):
    ei = edge_index.astype(jnp.int32)
    src = ei[0]
    dst = ei[1]
    z16 = jnp.zeros((N_PAD, HID), _f32)
    z1 = jnp.zeros((N_PAD,), _f32)

    g1, n1 = _mlp_in(x, W1, b1)

    beta1_vec = jnp.ones((16,), _f32)
    acc1, den1 = _conv_edges(g1, n1, src, dst, beta1_vec, z16, z1)
    g2, n2 = _combine(acc1, den1, g1, n1, jnp.ones((1, 1), _f32))

    b2f = beta2.astype(_f32)
    beta2_vec = jnp.broadcast_to(b2f, (16,))
    acc2, den2 = _conv_edges(g2, n2, src, dst, beta2_vec, z16, z1)
    out = _final(acc2, den2, g2, n2, b2f.reshape(1, 1), W2, b2)
    return out[:N_NODES]


# speedup vs baseline: 1.4684x; 1.4684x over previous
import functools

import jax
import jax.numpy as jnp
from jax import lax
from jax.experimental import pallas as pl
from jax.experimental.pallas import tpu as pltpu
from jax.experimental.pallas import tpu_sc as plsc

N_NODES = 100000
N_EDGES = 3200000
IN_DIM = 128
HID = 16
OUT_DIM = 128

NC = 2
NS = 16
NW = NC * NS
EPW = N_EDGES // NW
CHUNK = 80
NCHUNKS = EPW // CHUNK
N_PAD = 100352
ROWS_PER_TILE = N_PAD // NS

_f32 = jnp.float32


def _mlp_in_body(x_ref, w_ref, b_ref, h_ref):
    h = jnp.dot(x_ref[...], w_ref[...], preferred_element_type=_f32)
    h_ref[...] = jnp.maximum(h + b_ref[...], 0.0)


def _mlp_in(x, W1, b1):
    B = 2000
    return pl.pallas_call(
        _mlp_in_body,
        grid=(N_NODES // B,),
        in_specs=[
            pl.BlockSpec((B, IN_DIM), lambda i: (i, 0)),
            pl.BlockSpec((IN_DIM, HID), lambda i: (0, 0)),
            pl.BlockSpec((1, HID), lambda i: (0, 0)),
        ],
        out_specs=pl.BlockSpec((B, HID), lambda i: (i, 0)),
        out_shape=jax.ShapeDtypeStruct((N_PAD, HID), _f32),
    )(x, W1, b1.reshape(1, HID))


def _rsqrt16(x):
    i = plsc.bitcast(x, jnp.int32)
    i = jnp.int32(0x5F3759DF) - lax.shift_right_arithmetic(i, 1)
    y = plsc.bitcast(i, _f32)
    xh = x * 0.5
    for _ in range(3):
        y = y * (1.5 - xh * y * y)
    return y


def _conv_body(h_hbm, src_hbm, dst_hbm, beta_hbm, z16_hbm, z1_hbm,
               accp_hbm, denp_hbm,
               acc_sh, den_sh, betav, sidv, didv, srows, drows, wv, sem):
    cid = lax.axis_index("c")
    sid = lax.axis_index("s")
    wid = sid * NC + cid

    row0 = sid * ROWS_PER_TILE
    pltpu.sync_copy(z16_hbm.at[pl.ds(row0, ROWS_PER_TILE)],
                    acc_sh.at[pl.ds(row0, ROWS_PER_TILE)])
    pltpu.sync_copy(z1_hbm.at[pl.ds(row0, ROWS_PER_TILE)],
                    den_sh.at[pl.ds(row0, ROWS_PER_TILE)])
    pltpu.sync_copy(beta_hbm, betav)
    plsc.subcore_barrier()

    beta = betav[...]
    ebase = wid * EPW

    def chunk_body(i, carry):
        base = ebase + i * CHUNK
        pltpu.sync_copy(src_hbm.at[pl.ds(base, CHUNK)], sidv)
        pltpu.sync_copy(dst_hbm.at[pl.ds(base, CHUNK)], didv)
        cp_s = pltpu.async_copy(h_hbm.at[sidv], srows, sem)
        cp_d = pltpu.async_copy(h_hbm.at[didv], drows, sem)
        cp_s.wait()
        cp_d.wait()
        for g in range(CHUNK // 16):
            eidx = lax.iota(jnp.int32, 16) + g * 16
            dot = jnp.zeros((16,), _f32)
            ss = jnp.zeros((16,), _f32)
            dd = jnp.zeros((16,), _f32)
            scols = []
            for f in range(HID):
                fv = jnp.full((16,), f, jnp.int32)
                sf = plsc.load_gather(srows, [eidx, fv])
                df = plsc.load_gather(drows, [eidx, fv])
                scols.append(sf)
                dot = dot + sf * df
                ss = ss + sf * sf
                dd = dd + df * df
            rr = (_rsqrt16(jnp.maximum(ss, 1e-24))
                  * _rsqrt16(jnp.maximum(dd, 1e-24)))
            w = jnp.exp(dot * rr * beta)
            wv[pl.ds(g * 16, 16)] = w
            for f in range(HID):
                fv = jnp.full((16,), f, jnp.int32)
                plsc.store_scatter(srows, [eidx, fv], scols[f] * w)
        pltpu.sync_copy(srows, acc_sh.at[didv], add=True)
        pltpu.sync_copy(wv, den_sh.at[didv], add=True)
        return carry

    lax.fori_loop(0, NCHUNKS, chunk_body, 0)
    plsc.subcore_barrier()
    pltpu.sync_copy(acc_sh.at[pl.ds(row0, ROWS_PER_TILE)],
                    accp_hbm.at[cid, pl.ds(row0, ROWS_PER_TILE)])
    pltpu.sync_copy(den_sh.at[pl.ds(row0, ROWS_PER_TILE)],
                    denp_hbm.at[cid, 0, pl.ds(row0, ROWS_PER_TILE)])


def _conv_edges(h, src, dst, beta_vec, z16, z1):
    mesh = plsc.VectorSubcoreMesh(core_axis_name="c", subcore_axis_name="s")
    fn = pl.kernel(
        _conv_body,
        mesh=mesh,
        compiler_params=pltpu.CompilerParams(
            needs_layout_passes=False, use_tc_tiling_on_sc=False),
        out_type=[
            jax.ShapeDtypeStruct((NC, N_PAD, HID), _f32),
            jax.ShapeDtypeStruct((NC, 1, N_PAD), _f32),
        ],
        scratch_types=[
            pltpu.VMEM_SHARED((N_PAD, HID), _f32),
            pltpu.VMEM_SHARED((N_PAD,), _f32),
            pltpu.VMEM((16,), _f32),
            pltpu.VMEM((CHUNK,), jnp.int32),
            pltpu.VMEM((CHUNK,), jnp.int32),
            pltpu.VMEM((CHUNK, HID), _f32),
            pltpu.VMEM((CHUNK, HID), _f32),
            pltpu.VMEM((CHUNK,), _f32),
            pltpu.SemaphoreType.DMA,
        ],
    )
    return fn(h, src, dst, beta_vec, z16, z1)


def _combine_core(accp_ref, denp_ref, h_ref, beta_ref):
    h = h_ref[...]
    n2 = jnp.sum(h * h, axis=1, keepdims=True)
    rr = 1.0 / jnp.maximum(jnp.sqrt(n2), 1e-12)
    beta = beta_ref[0, 0]
    wl = jnp.exp(n2 * rr * rr * beta)
    num = accp_ref[0] + accp_ref[1] + wl * h
    den = denp_ref[0, 0] + denp_ref[1, 0] + wl[:, 0]
    return num / jnp.maximum(den, 1e-16)[:, None]


def _combine_body(accp_ref, denp_ref, h_ref, beta_ref, h2_ref):
    h2_ref[...] = _combine_core(accp_ref, denp_ref, h_ref, beta_ref)


def _final_body(accp_ref, denp_ref, h_ref, beta_ref, w2_ref, b2_ref, o_ref):
    h2 = _combine_core(accp_ref, denp_ref, h_ref, beta_ref)
    o = jnp.dot(h2, w2_ref[...], preferred_element_type=_f32) + b2_ref[...]
    o_ref[...] = jnp.tanh(o)


def _combine(accp, denp, h, beta11):
    B = 2048
    return pl.pallas_call(
        _combine_body,
        grid=(N_PAD // B,),
        in_specs=[
            pl.BlockSpec((NC, B, HID), lambda i: (0, i, 0)),
            pl.BlockSpec((NC, 1, B), lambda i: (0, 0, i)),
            pl.BlockSpec((B, HID), lambda i: (i, 0)),
            pl.BlockSpec((1, 1), lambda i: (0, 0)),
        ],
        out_specs=pl.BlockSpec((B, HID), lambda i: (i, 0)),
        out_shape=jax.ShapeDtypeStruct((N_PAD, HID), _f32),
    )(accp, denp, h, beta11)


def _final(accp, denp, h, beta11, W2, b2):
    B = 2048
    return pl.pallas_call(
        _final_body,
        grid=(N_PAD // B,),
        in_specs=[
            pl.BlockSpec((NC, B, HID), lambda i: (0, i, 0)),
            pl.BlockSpec((NC, 1, B), lambda i: (0, 0, i)),
            pl.BlockSpec((B, HID), lambda i: (i, 0)),
            pl.BlockSpec((1, 1), lambda i: (0, 0)),
            pl.BlockSpec((HID, OUT_DIM), lambda i: (0, 0)),
            pl.BlockSpec((1, OUT_DIM), lambda i: (0, 0)),
        ],
        out_specs=pl.BlockSpec((B, OUT_DIM), lambda i: (i, 0)),
        out_shape=jax.ShapeDtypeStruct((N_PAD, OUT_DIM), _f32),
    )(accp, denp, h, beta11, W2, b2.reshape(1, OUT_DIM))


def kernel(x, edge_index, W1, b1, W2, b2, beta2):
    ei = edge_index.astype(jnp.int32)
    src = ei[0]
    dst = ei[1]
    z16 = jnp.zeros((N_PAD, HID), _f32)
    z1 = jnp.zeros((N_PAD,), _f32)

    h1 = _mlp_in(x, W1, b1)

    beta1_vec = jnp.ones((16,), _f32)
    acc1, den1 = _conv_edges(h1, src, dst, beta1_vec, z16, z1)
    h2 = _combine(acc1, den1, h1, jnp.ones((1, 1), _f32))

    b2f = beta2.astype(_f32)
    beta2_vec = jnp.broadcast_to(b2f, (16,))
    acc2, den2 = _conv_edges(h2, src, dst, beta2_vec, z16, z1)
    out = _final(acc2, den2, h2, b2f.reshape(1, 1), W2, b2)
    return out[:N_NODES]


# speedup vs baseline: 2.1674x; 1.4760x over previous
import functools

import jax
import jax.numpy as jnp
from jax import lax
from jax.experimental import pallas as pl
from jax.experimental.pallas import tpu as pltpu
from jax.experimental.pallas import tpu_sc as plsc

N_NODES = 100000
N_EDGES = 3200000
IN_DIM = 128
HID = 16
OUT_DIM = 128

NC = 2
NS = 16
NW = NC * NS
EPW = N_EDGES // NW
CHUNK = 80
NCHUNKS = EPW // CHUNK
N_PAD = 100352
ROWS_PER_TILE = N_PAD // NS

_f32 = jnp.float32


def _mlp_in_body(x_ref, w_ref, b_ref, h_ref):
    h = jnp.dot(x_ref[...], w_ref[...], preferred_element_type=_f32)
    h_ref[...] = jnp.maximum(h + b_ref[...], 0.0)


def _mlp_in(x, W1, b1):
    B = 2000
    return pl.pallas_call(
        _mlp_in_body,
        grid=(N_NODES // B,),
        in_specs=[
            pl.BlockSpec((B, IN_DIM), lambda i: (i, 0)),
            pl.BlockSpec((IN_DIM, HID), lambda i: (0, 0)),
            pl.BlockSpec((1, HID), lambda i: (0, 0)),
        ],
        out_specs=pl.BlockSpec((B, HID), lambda i: (i, 0)),
        out_shape=jax.ShapeDtypeStruct((N_PAD, HID), _f32),
    )(x, W1, b1.reshape(1, HID))


def _rsqrt16(x):
    i = plsc.bitcast(x, jnp.int32)
    i = jnp.int32(0x5F3759DF) - lax.shift_right_arithmetic(i, 1)
    y = plsc.bitcast(i, _f32)
    xh = x * 0.5
    for _ in range(3):
        y = y * (1.5 - xh * y * y)
    return y


def _conv_body(h_hbm, idx2_hbm, beta_hbm, z16_hbm, z1_hbm,
               accp_hbm, denp_hbm,
               acc_sh, den_sh, betav,
               idxA, idxB, srowsA, drowsA, srowsB, drowsB, wv,
               semA, semB):
    cid = lax.axis_index("c")
    sid = lax.axis_index("s")
    wid = sid * NC + cid

    row0 = sid * ROWS_PER_TILE
    pltpu.sync_copy(z16_hbm.at[pl.ds(row0, ROWS_PER_TILE)],
                    acc_sh.at[pl.ds(row0, ROWS_PER_TILE)])
    pltpu.sync_copy(z1_hbm.at[pl.ds(row0, ROWS_PER_TILE)],
                    den_sh.at[pl.ds(row0, ROWS_PER_TILE)])
    pltpu.sync_copy(beta_hbm, betav)
    plsc.subcore_barrier()

    beta = betav[...]
    cbase = wid * NCHUNKS

    def issue(idxv, srows, drows, sem):
        pltpu.async_copy(h_hbm.at[idxv.at[0]], srows, sem)
        pltpu.async_copy(h_hbm.at[idxv.at[1]], drows, sem)

    def drain(srows, drows, sem):
        pltpu.make_async_copy(h_hbm.at[pl.ds(0, CHUNK)], srows, sem).wait()
        pltpu.make_async_copy(h_hbm.at[pl.ds(0, CHUNK)], drows, sem).wait()

    def compute_scatter(idxv, srows, drows):
        for g in range(CHUNK // 16):
            eidx = lax.iota(jnp.int32, 16) + g * 16
            dot = jnp.zeros((16,), _f32)
            ss = jnp.zeros((16,), _f32)
            dd = jnp.zeros((16,), _f32)
            scols = []
            for f in range(HID):
                fv = jnp.full((16,), f, jnp.int32)
                sf = plsc.load_gather(srows, [eidx, fv])
                df = plsc.load_gather(drows, [eidx, fv])
                scols.append(sf)
                dot = dot + sf * df
                ss = ss + sf * sf
                dd = dd + df * df
            rr = (_rsqrt16(jnp.maximum(ss, 1e-24))
                  * _rsqrt16(jnp.maximum(dd, 1e-24)))
            w = jnp.exp(dot * rr * beta)
            wv[pl.ds(g * 16, 16)] = w
            for f in range(HID):
                fv = jnp.full((16,), f, jnp.int32)
                plsc.store_scatter(srows, [eidx, fv], scols[f] * w)
        pltpu.sync_copy(srows, acc_sh.at[idxv.at[1]], add=True)
        pltpu.sync_copy(wv, den_sh.at[idxv.at[1]], add=True)

    pltpu.sync_copy(idx2_hbm.at[cbase], idxA)
    issue(idxA, srowsA, drowsA, semA)

    def loop_body(g, carry):
        c0 = 2 * g
        pltpu.sync_copy(idx2_hbm.at[cbase + c0 + 1], idxB)
        issue(idxB, srowsB, drowsB, semB)
        drain(srowsA, drowsA, semA)
        compute_scatter(idxA, srowsA, drowsA)
        @pl.when(c0 + 2 < NCHUNKS)
        def _():
            pltpu.sync_copy(idx2_hbm.at[cbase + c0 + 2], idxA)
            issue(idxA, srowsA, drowsA, semA)
        drain(srowsB, drowsB, semB)
        compute_scatter(idxB, srowsB, drowsB)
        return carry

    lax.fori_loop(0, NCHUNKS // 2, loop_body, 0)
    plsc.subcore_barrier()
    pltpu.sync_copy(acc_sh.at[pl.ds(row0, ROWS_PER_TILE)],
                    accp_hbm.at[cid, pl.ds(row0, ROWS_PER_TILE)])
    pltpu.sync_copy(den_sh.at[pl.ds(row0, ROWS_PER_TILE)],
                    denp_hbm.at[cid, 0, pl.ds(row0, ROWS_PER_TILE)])


def _conv_edges(h, idx2, beta_vec, z16, z1):
    mesh = plsc.VectorSubcoreMesh(core_axis_name="c", subcore_axis_name="s")
    fn = pl.kernel(
        _conv_body,
        mesh=mesh,
        compiler_params=pltpu.CompilerParams(
            needs_layout_passes=False, use_tc_tiling_on_sc=False),
        out_type=[
            jax.ShapeDtypeStruct((NC, N_PAD, HID), _f32),
            jax.ShapeDtypeStruct((NC, 1, N_PAD), _f32),
        ],
        scratch_types=[
            pltpu.VMEM_SHARED((N_PAD, HID), _f32),
            pltpu.VMEM_SHARED((N_PAD,), _f32),
            pltpu.VMEM((16,), _f32),
            pltpu.VMEM((2, CHUNK), jnp.int32),
            pltpu.VMEM((2, CHUNK), jnp.int32),
            pltpu.VMEM((CHUNK, HID), _f32),
            pltpu.VMEM((CHUNK, HID), _f32),
            pltpu.VMEM((CHUNK, HID), _f32),
            pltpu.VMEM((CHUNK, HID), _f32),
            pltpu.VMEM((CHUNK,), _f32),
            pltpu.SemaphoreType.DMA,
            pltpu.SemaphoreType.DMA,
        ],
    )
    return fn(h, idx2, beta_vec, z16, z1)


def _combine_core(accp_ref, denp_ref, h_ref, beta_ref):
    h = h_ref[...]
    n2 = jnp.sum(h * h, axis=1, keepdims=True)
    rr = 1.0 / jnp.maximum(jnp.sqrt(n2), 1e-12)
    beta = beta_ref[0, 0]
    wl = jnp.exp(n2 * rr * rr * beta)
    num = accp_ref[0] + accp_ref[1] + wl * h
    den = denp_ref[0, 0] + denp_ref[1, 0] + wl[:, 0]
    return num / jnp.maximum(den, 1e-16)[:, None]


def _combine_body(accp_ref, denp_ref, h_ref, beta_ref, h2_ref):
    h2_ref[...] = _combine_core(accp_ref, denp_ref, h_ref, beta_ref)


def _final_body(accp_ref, denp_ref, h_ref, beta_ref, w2_ref, b2_ref, o_ref):
    h2 = _combine_core(accp_ref, denp_ref, h_ref, beta_ref)
    o = jnp.dot(h2, w2_ref[...], preferred_element_type=_f32) + b2_ref[...]
    o_ref[...] = jnp.tanh(o)


def _combine(accp, denp, h, beta11):
    B = 2048
    return pl.pallas_call(
        _combine_body,
        grid=(N_PAD // B,),
        in_specs=[
            pl.BlockSpec((NC, B, HID), lambda i: (0, i, 0)),
            pl.BlockSpec((NC, 1, B), lambda i: (0, 0, i)),
            pl.BlockSpec((B, HID), lambda i: (i, 0)),
            pl.BlockSpec((1, 1), lambda i: (0, 0)),
        ],
        out_specs=pl.BlockSpec((B, HID), lambda i: (i, 0)),
        out_shape=jax.ShapeDtypeStruct((N_PAD, HID), _f32),
    )(accp, denp, h, beta11)


def _final(accp, denp, h, beta11, W2, b2):
    B = 2048
    return pl.pallas_call(
        _final_body,
        grid=(N_PAD // B,),
        in_specs=[
            pl.BlockSpec((NC, B, HID), lambda i: (0, i, 0)),
            pl.BlockSpec((NC, 1, B), lambda i: (0, 0, i)),
            pl.BlockSpec((B, HID), lambda i: (i, 0)),
            pl.BlockSpec((1, 1), lambda i: (0, 0)),
            pl.BlockSpec((HID, OUT_DIM), lambda i: (0, 0)),
            pl.BlockSpec((1, OUT_DIM), lambda i: (0, 0)),
        ],
        out_specs=pl.BlockSpec((B, OUT_DIM), lambda i: (i, 0)),
        out_shape=jax.ShapeDtypeStruct((N_PAD, OUT_DIM), _f32),
    )(accp, denp, h, beta11, W2, b2.reshape(1, OUT_DIM))


def kernel(x, edge_index, W1, b1, W2, b2, beta2):
    ei = edge_index.astype(jnp.int32)
    idx2 = jnp.stack([ei[0].reshape(-1, CHUNK), ei[1].reshape(-1, CHUNK)],
                     axis=1)
    z16 = jnp.zeros((N_PAD, HID), _f32)
    z1 = jnp.zeros((N_PAD,), _f32)

    h1 = _mlp_in(x, W1, b1)

    beta1_vec = jnp.ones((16,), _f32)
    acc1, den1 = _conv_edges(h1, idx2, beta1_vec, z16, z1)
    h2 = _combine(acc1, den1, h1, jnp.ones((1, 1), _f32))

    b2f = beta2.astype(_f32)
    beta2_vec = jnp.broadcast_to(b2f, (16,))
    acc2, den2 = _conv_edges(h2, idx2, beta2_vec, z16, z1)
    out = _final(acc2, den2, h2, b2f.reshape(1, 1), W2, b2)
    return out[:N_NODES]


# speedup vs baseline: 2.7089x; 1.2499x over previous
import functools

import jax
import jax.numpy as jnp
from jax import lax
from jax.experimental import pallas as pl
from jax.experimental.pallas import tpu as pltpu
from jax.experimental.pallas import tpu_sc as plsc

N_NODES = 100000
N_EDGES = 3200000
IN_DIM = 128
HID = 16
OUT_DIM = 128

NC = 2
NS = 16
NW = NC * NS
EPW = N_EDGES // NW
CHUNK = 80
NCHUNKS = EPW // CHUNK
N_PAD = 100352
ROWS_PER_TILE = N_PAD // NS

_f32 = jnp.float32


def _mlp_in_body(x_ref, w_ref, b_ref, h_ref):
    h = jnp.dot(x_ref[...], w_ref[...], preferred_element_type=_f32)
    h_ref[...] = jnp.maximum(h + b_ref[...], 0.0)


def _mlp_in(x, W1, b1):
    B = 2000
    return pl.pallas_call(
        _mlp_in_body,
        grid=(N_NODES // B,),
        in_specs=[
            pl.BlockSpec((B, IN_DIM), lambda i: (i, 0)),
            pl.BlockSpec((IN_DIM, HID), lambda i: (0, 0)),
            pl.BlockSpec((1, HID), lambda i: (0, 0)),
        ],
        out_specs=pl.BlockSpec((B, HID), lambda i: (i, 0)),
        out_shape=jax.ShapeDtypeStruct((N_PAD, HID), _f32),
    )(x, W1, b1.reshape(1, HID))


def _rsqrt16(x):
    i = plsc.bitcast(x, jnp.int32)
    i = jnp.int32(0x5F3759DF) - lax.shift_right_arithmetic(i, 1)
    y = plsc.bitcast(i, _f32)
    xh = x * 0.5
    for _ in range(3):
        y = y * (1.5 - xh * y * y)
    return y


def _conv_body(h_hbm, idx2_hbm, beta_hbm, z16_hbm, z1_hbm,
               accp_hbm, denp_hbm,
               acc_sh, den_sh, betav,
               i0, i1, i2, i3, s0, d0, s1, d1, s2, d2, s3, d3, wv,
               sr0, sr1, sr2, sr3, si0, si1):
    cid = lax.axis_index("c")
    sid = lax.axis_index("s")
    wid = sid * NC + cid

    row0 = sid * ROWS_PER_TILE
    pltpu.sync_copy(z16_hbm.at[pl.ds(row0, ROWS_PER_TILE)],
                    acc_sh.at[pl.ds(row0, ROWS_PER_TILE)])
    pltpu.sync_copy(z1_hbm.at[pl.ds(row0, ROWS_PER_TILE)],
                    den_sh.at[pl.ds(row0, ROWS_PER_TILE)])
    pltpu.sync_copy(beta_hbm, betav)
    plsc.subcore_barrier()

    beta = betav[...]
    cbase = wid * NCHUNKS

    I = [i0, i1, i2, i3]
    S = [s0, s1, s2, s3]
    D = [d0, d1, d2, d3]
    SR = [sr0, sr1, sr2, sr3]
    SI = [si0, si1]

    def issue_rows(k):
        pltpu.async_copy(h_hbm.at[I[k].at[0]], S[k], SR[k])
        pltpu.async_copy(h_hbm.at[I[k].at[1]], D[k], SR[k])

    def drain_rows(k):
        pltpu.make_async_copy(h_hbm.at[pl.ds(0, CHUNK)], S[k], SR[k]).wait()
        pltpu.make_async_copy(h_hbm.at[pl.ds(0, CHUNK)], D[k], SR[k]).wait()

    def wait_idx(p, k):
        pltpu.make_async_copy(idx2_hbm.at[0], I[k], SI[p]).wait()

    def compute_scatter(idxv, srows, drows):
        for g in range(CHUNK // 16):
            eidx = lax.iota(jnp.int32, 16) + g * 16
            dot = jnp.zeros((16,), _f32)
            ss = jnp.zeros((16,), _f32)
            dd = jnp.zeros((16,), _f32)
            scols = []
            for f in range(HID):
                fv = jnp.full((16,), f, jnp.int32)
                sf = plsc.load_gather(srows, [eidx, fv])
                df = plsc.load_gather(drows, [eidx, fv])
                scols.append(sf)
                dot = dot + sf * df
                ss = ss + sf * sf
                dd = dd + df * df
            rr = (_rsqrt16(jnp.maximum(ss, 1e-24))
                  * _rsqrt16(jnp.maximum(dd, 1e-24)))
            w = jnp.exp(dot * rr * beta)
            wv[pl.ds(g * 16, 16)] = w
            for f in range(HID):
                fv = jnp.full((16,), f, jnp.int32)
                plsc.store_scatter(srows, [eidx, fv], scols[f] * w)
        pltpu.sync_copy(srows, acc_sh.at[idxv.at[1]], add=True)
        pltpu.sync_copy(wv, den_sh.at[idxv.at[1]], add=True)

    pltpu.sync_copy(idx2_hbm.at[cbase], i0)
    pltpu.sync_copy(idx2_hbm.at[cbase + 1], i1)
    issue_rows(0)
    issue_rows(1)
    pltpu.async_copy(idx2_hbm.at[cbase + 2], i2, si0)
    pltpu.async_copy(idx2_hbm.at[cbase + 3], i3, si1)

    def slot(c, k):
        p = k % 2
        k2 = (k + 2) % 4
        wait_idx(p, k2)
        issue_rows(k2)
        drain_rows(k)
        compute_scatter(I[k], S[k], D[k])

        @pl.when(c + 4 < NCHUNKS)
        def _():
            pltpu.async_copy(idx2_hbm.at[cbase + c + 4], I[k], SI[p])

    def loop_body(g, carry):
        c0 = 4 * g
        for k in range(4):
            slot(c0 + k, k)
        return carry

    lax.fori_loop(0, (NCHUNKS - 2) // 4, loop_body, 0)
    drain_rows(0)
    compute_scatter(i0, s0, d0)
    drain_rows(1)
    compute_scatter(i1, s1, d1)
    plsc.subcore_barrier()
    pltpu.sync_copy(acc_sh.at[pl.ds(row0, ROWS_PER_TILE)],
                    accp_hbm.at[cid, pl.ds(row0, ROWS_PER_TILE)])
    pltpu.sync_copy(den_sh.at[pl.ds(row0, ROWS_PER_TILE)],
                    denp_hbm.at[cid, 0, pl.ds(row0, ROWS_PER_TILE)])


def _conv_edges(h, idx2, beta_vec, z16, z1):
    mesh = plsc.VectorSubcoreMesh(core_axis_name="c", subcore_axis_name="s")
    fn = pl.kernel(
        _conv_body,
        mesh=mesh,
        compiler_params=pltpu.CompilerParams(
            needs_layout_passes=False, use_tc_tiling_on_sc=False),
        out_type=[
            jax.ShapeDtypeStruct((NC, N_PAD, HID), _f32),
            jax.ShapeDtypeStruct((NC, 1, N_PAD), _f32),
        ],
        scratch_types=(
            [pltpu.VMEM_SHARED((N_PAD, HID), _f32),
             pltpu.VMEM_SHARED((N_PAD,), _f32),
             pltpu.VMEM((16,), _f32)]
            + [pltpu.VMEM((2, CHUNK), jnp.int32)] * 4
            + [pltpu.VMEM((CHUNK, HID), _f32)] * 8
            + [pltpu.VMEM((CHUNK,), _f32)]
            + [pltpu.SemaphoreType.DMA] * 6
        ),
    )
    return fn(h, idx2, beta_vec, z16, z1)


def _combine_core(accp_ref, denp_ref, h_ref, beta_ref):
    h = h_ref[...]
    n2 = jnp.sum(h * h, axis=1, keepdims=True)
    rr = 1.0 / jnp.maximum(jnp.sqrt(n2), 1e-12)
    beta = beta_ref[0, 0]
    wl = jnp.exp(n2 * rr * rr * beta)
    num = accp_ref[0] + accp_ref[1] + wl * h
    den = denp_ref[0, 0] + denp_ref[1, 0] + wl[:, 0]
    return num / jnp.maximum(den, 1e-16)[:, None]


def _combine_body(accp_ref, denp_ref, h_ref, beta_ref, h2_ref):
    h2_ref[...] = _combine_core(accp_ref, denp_ref, h_ref, beta_ref)


def _final_body(accp_ref, denp_ref, h_ref, beta_ref, w2_ref, b2_ref, o_ref):
    h2 = _combine_core(accp_ref, denp_ref, h_ref, beta_ref)
    o = jnp.dot(h2, w2_ref[...], preferred_element_type=_f32) + b2_ref[...]
    o_ref[...] = jnp.tanh(o)


def _combine(accp, denp, h, beta11):
    B = 2048
    return pl.pallas_call(
        _combine_body,
        grid=(N_PAD // B,),
        in_specs=[
            pl.BlockSpec((NC, B, HID), lambda i: (0, i, 0)),
            pl.BlockSpec((NC, 1, B), lambda i: (0, 0, i)),
            pl.BlockSpec((B, HID), lambda i: (i, 0)),
            pl.BlockSpec((1, 1), lambda i: (0, 0)),
        ],
        out_specs=pl.BlockSpec((B, HID), lambda i: (i, 0)),
        out_shape=jax.ShapeDtypeStruct((N_PAD, HID), _f32),
    )(accp, denp, h, beta11)


def _final(accp, denp, h, beta11, W2, b2):
    B = 2048
    return pl.pallas_call(
        _final_body,
        grid=(N_PAD // B,),
        in_specs=[
            pl.BlockSpec((NC, B, HID), lambda i: (0, i, 0)),
            pl.BlockSpec((NC, 1, B), lambda i: (0, 0, i)),
            pl.BlockSpec((B, HID), lambda i: (i, 0)),
            pl.BlockSpec((1, 1), lambda i: (0, 0)),
            pl.BlockSpec((HID, OUT_DIM), lambda i: (0, 0)),
            pl.BlockSpec((1, OUT_DIM), lambda i: (0, 0)),
        ],
        out_specs=pl.BlockSpec((B, OUT_DIM), lambda i: (i, 0)),
        out_shape=jax.ShapeDtypeStruct((N_PAD, OUT_DIM), _f32),
    )(accp, denp, h, beta11, W2, b2.reshape(1, OUT_DIM))


def kernel(x, edge_index, W1, b1, W2, b2, beta2):
    ei = edge_index.astype(jnp.int32)
    idx2 = jnp.stack([ei[0].reshape(-1, CHUNK), ei[1].reshape(-1, CHUNK)],
                     axis=1)
    z16 = jnp.zeros((N_PAD, HID), _f32)
    z1 = jnp.zeros((N_PAD,), _f32)

    h1 = _mlp_in(x, W1, b1)

    beta1_vec = jnp.ones((16,), _f32)
    acc1, den1 = _conv_edges(h1, idx2, beta1_vec, z16, z1)
    h2 = _combine(acc1, den1, h1, jnp.ones((1, 1), _f32))

    b2f = beta2.astype(_f32)
    beta2_vec = jnp.broadcast_to(b2f, (16,))
    acc2, den2 = _conv_edges(h2, idx2, beta2_vec, z16, z1)
    out = _final(acc2, den2, h2, b2f.reshape(1, 1), W2, b2)
    return out[:N_NODES]
